# baseline (speedup 1.0000x reference)
import functools

import jax
import jax.numpy as jnp
from jax import lax
from jax.experimental import pallas as pl
from jax.experimental.pallas import tpu as pltpu
from jax.experimental.pallas import tpu_sc as plsc

B, N, D = 1024, 50, 512
NUM_TYPES = 4
NC, NS, L = 2, 16, 16
NW = NC * NS
BPW = B // NW
N_PAD = 72
E_PAD = 8
NB = 4


def _make_kernel():
    mesh = plsc.VectorSubcoreMesh(core_axis_name="c", subcore_axis_name="s")

    @functools.partial(
        pl.kernel,
        mesh=mesh,
        out_type=jax.ShapeDtypeStruct((B, N, D), jnp.float32),
        scratch_types=[
            pltpu.VMEM((N_PAD,), jnp.int32),
            pltpu.VMEM((E_PAD, D), jnp.float32),
        ]
        + [pltpu.VMEM((N, D), jnp.float32) for _ in range(NB)]
        + [pltpu.SemaphoreType.DMA for _ in range(2 * NB)],
    )
    def k(cs_hbm, tid_hbm, emb_hbm, out_hbm, tid_v, emb_v, *rest):
        bufs = rest[:NB]
        isems = rest[NB:2 * NB]
        osems = rest[2 * NB:3 * NB]

        wid = lax.axis_index("s") * NC + lax.axis_index("c")
        base = wid * BPW

        pltpu.sync_copy(tid_hbm, tid_v)
        pltpu.sync_copy(emb_hbm, emb_v)

        def in_copy(t, p):
            return pltpu.make_async_copy(cs_hbm.at[base + t], bufs[p], isems[p])

        def out_copy(t, p):
            return pltpu.make_async_copy(bufs[p], out_hbm.at[base + t], osems[p])

        def compute(p):
            buf = bufs[p]

            def row_body(i, c):
                tv = tid_v[pl.ds(i, L)][0]
                for j in range(D // L):
                    sl = pl.ds(j * L, L)
                    buf[i, sl] = buf[i, sl] + emb_v[tv, sl]
                return c

            lax.fori_loop(0, N, row_body, 0)

        in_copy(0, 0).start()
        in_copy(1, 1).start()

        def step(t0, carry):
            for p in range(NB):
                t = t0 + p
                in_copy(t, p).wait()

                @pl.when(t + 2 < BPW)
                def _():
                    pf = (p + 2) % NB

                    @pl.when(t >= 2)
                    def _():
                        out_copy(t - 2, pf).wait()

                    in_copy(t + 2, pf).start()

                compute(p)
                out_copy(t, p).start()
            return carry

        lax.fori_loop(0, BPW // NB, lambda s, c: step(s * NB, c), 0)

        for t in range(BPW - NB, BPW):
            out_copy(t, t % NB).wait()

    return k


_k = _make_kernel()


def kernel(channel_stack, type_ids, embeddings):
    tid = jnp.zeros((N_PAD,), jnp.int32).at[:N].set(type_ids.astype(jnp.int32))
    emb = jnp.zeros((E_PAD, D), jnp.float32).at[:NUM_TYPES].set(embeddings)
    return _k(channel_stack, tid, emb)


# speedup vs baseline: 1.6370x; 1.6370x over previous
import functools

import jax
import jax.numpy as jnp
from jax import lax
from jax.experimental import pallas as pl
from jax.experimental.pallas import tpu as pltpu

B, N, D = 1024, 50, 512
NUM_TYPES = 4
BB = 16


def _add_body(ids_ref, emb_ref, x_ref, o_ref):
    tid = ids_ref[...]
    oh = (tid == lax.broadcasted_iota(jnp.int32, (N, NUM_TYPES), 1))
    temb = jnp.dot(oh.astype(jnp.float32), emb_ref[...],
                   preferred_element_type=jnp.float32)
    o_ref[...] = x_ref[...] + temb[None]


def kernel(channel_stack, type_ids, embeddings):
    ids2 = type_ids.astype(jnp.int32).reshape(N, 1)
    grid = (B // BB,)
    return pl.pallas_call(
        _add_body,
        grid=grid,
        in_specs=[
            pl.BlockSpec((N, 1), lambda i: (0, 0)),
            pl.BlockSpec((NUM_TYPES, D), lambda i: (0, 0)),
            pl.BlockSpec((BB, N, D), lambda i: (i, 0, 0)),
        ],
        out_specs=pl.BlockSpec((BB, N, D), lambda i: (i, 0, 0)),
        out_shape=jax.ShapeDtypeStruct((B, N, D), jnp.float32),
    )(ids2, embeddings, channel_stack)
